# Initial kernel scaffold; baseline (speedup 1.0000x reference)
import jax
import jax.numpy as jnp
from jax.experimental import pallas as pl


def kernel(node_feats, neighbor_feats, weight, node_count):
    raise NotImplementedError("write your pallas kernel here")


# speedup vs baseline: 1.1625x; 1.1625x over previous
import jax
import jax.numpy as jnp
from jax.experimental import pallas as pl
from jax.experimental.pallas import tpu as pltpu


def _fused_body(nf_ref, nb_ref, w_ref, out_ref, *, inv_ns):
    nsum = jnp.sum(nb_ref[...], axis=1)
    d = nf_ref.shape[1]
    w1 = w_ref[:d, :]
    w2 = w_ref[d:, :]
    acc = jnp.dot(nf_ref[...], w1, preferred_element_type=jnp.float32)
    acc += jnp.dot(nsum * inv_ns, w2, preferred_element_type=jnp.float32)
    out_ref[...] = jnp.maximum(acc, 0.0)


def kernel(node_feats, neighbor_feats, weight, node_count):
    n, d = node_feats.shape
    ns = neighbor_feats.shape[0] // n
    e = weight.shape[1]
    nb3 = neighbor_feats.reshape(n, ns, d)

    blk = 400
    grid = (n // blk,)

    import functools
    body = functools.partial(_fused_body, inv_ns=1.0 / ns)

    out = pl.pallas_call(
        body,
        grid=grid,
        in_specs=[
            pl.BlockSpec((blk, d), lambda i: (i, 0)),
            pl.BlockSpec((blk, ns, d), lambda i: (i, 0, 0)),
            pl.BlockSpec((2 * d, e), lambda i: (0, 0)),
        ],
        out_specs=pl.BlockSpec((blk, e), lambda i: (i, 0)),
        out_shape=jax.ShapeDtypeStruct((n, e), jnp.float32),
        compiler_params=pltpu.CompilerParams(
            dimension_semantics=("arbitrary",),
        ),
    )(node_feats, nb3, weight)
    return out
